# baseline (speedup 1.0000x reference)
import functools

import jax
import jax.numpy as jnp
from jax.experimental import pallas as pl

N = 10000
E_EDGES = 160000
HID = 256


def _linear_kernel(x_ref, w_ref, b_ref, o_ref):
    o_ref[...] = (
        jnp.dot(x_ref[...], w_ref[...], preferred_element_type=jnp.float32)
        + b_ref[...]
    )


def _linear(x, w, b, block_m=512):
    m, k = x.shape
    f = w.shape[1]
    grid = (pl.cdiv(m, block_m),)
    return pl.pallas_call(
        _linear_kernel,
        grid=grid,
        in_specs=[
            pl.BlockSpec((block_m, k), lambda i: (i, 0)),
            pl.BlockSpec((k, f), lambda i: (0, 0)),
            pl.BlockSpec((f,), lambda i: (0,)),
        ],
        out_specs=pl.BlockSpec((block_m, f), lambda i: (i, 0)),
        out_shape=jax.ShapeDtypeStruct((m, f), jnp.float32),
    )(x, w, b)


def _bn(x, g, b):
    m = x.mean(axis=0)
    v = x.var(axis=0)
    return (x - m) / jnp.sqrt(v + 1e-5) * g + b


def _gated_layer(p, h, e, src, dst, snorm_n, snorm_e):
    h_in, e_in = h, e
    w_abde = jnp.concatenate([p["A"]["w"], p["B"]["w"], p["D"]["w"], p["E"]["w"]], axis=1)
    b_abde = jnp.concatenate([p["A"]["b"], p["B"]["b"], p["D"]["b"], p["E"]["b"]], axis=0)
    abde = _linear(h, w_abde, b_abde, block_m=400)
    Ah = abde[:, :HID]
    Bh = abde[:, HID : 2 * HID]
    Dh = abde[:, 2 * HID : 3 * HID]
    Eh = abde[:, 3 * HID :]
    Ce = _linear(e, p["C"]["w"], p["C"]["b"], block_m=640)
    e_new = Dh[src] + Eh[dst] + Ce
    sigma = jax.nn.sigmoid(e_new)
    num = jax.ops.segment_sum(sigma * Bh[src], dst, num_segments=N)
    den = jax.ops.segment_sum(sigma, dst, num_segments=N)
    h_new = Ah + num / (den + 1e-6)
    h_new = h_new * snorm_n
    e_new = e_new * snorm_e
    h_new = _bn(h_new, p["bn_h_g"], p["bn_h_b"])
    e_new = _bn(e_new, p["bn_e_g"], p["bn_e_b"])
    h_new = jax.nn.relu(h_new)
    e_new = jax.nn.relu(e_new)
    return h_in + h_new, e_in + e_new


def kernel(h, e, edge_index, snorm_n, snorm_e, label, delta, params):
    src = edge_index[0]
    dst = edge_index[1]
    hv = params["emb_h"][h]
    ev = e * params["emb_e"]["w"][0] + params["emb_e"]["b"]
    for p in params["layers"]:
        hv, ev = _gated_layer(p, hv, ev, src, dst, snorm_n, snorm_e)
    x = hv
    nmlp = len(params["mlp"])
    for i, lin in enumerate(params["mlp"]):
        x = _linear(x, lin["w"], lin["b"], block_m=2000)
        if i < nmlp - 1:
            x = jax.nn.relu(x)
    p_out = x
    hc = jnp.concatenate([hv, label], axis=1)
    w = jax.nn.sigmoid(_linear(hc, params["mlp2"]["w"], params["mlp2"]["b"], block_m=2000))
    w = jnp.tile(w, (1, label.shape[1]))
    w = jnp.clip(w, 0.0, jnp.asarray(delta, dtype=jnp.float32))
    ones = jnp.ones_like(label)
    max_entropy = jnp.full_like(label, 1.0 / label.shape[1])
    g_hat = (ones - w) * label + w * max_entropy
    return p_out, g_hat
